# Initial kernel scaffold; baseline (speedup 1.0000x reference)
import jax
import jax.numpy as jnp
from jax.experimental import pallas as pl


def kernel(W_ent, W_rel, h_ids, r_typ, t_ids):
    raise NotImplementedError("write your pallas kernel here")


# speedup vs baseline: 7.1709x; 7.1709x over previous
import jax
import jax.numpy as jnp
from jax import lax
from jax.experimental import pallas as pl
from jax.experimental.pallas import tpu as pltpu
from jax.experimental.pallas import tpu_sc as plsc

ENT_N = 100000
REL_N = 237
DIM = 64
N = 16384

NC = 2
NS = 16
NW = NC * NS
MLOC = (REL_N + NW - 1) // NW
CHUNK = 64
LANES = 16


def _rsqrt(x):
    yi = jnp.int32(0x5F3759DF) - lax.shift_right_logical(
        plsc.bitcast(x, jnp.int32), 1)
    y = plsc.bitcast(yi, jnp.float32)
    for _ in range(3):
        y = y * (1.5 - 0.5 * x * y * y)
    return y


def _body(went, hids, rtyp, tids, out,
          rt_v, hid_v, tid_v, midx_v, rstage_v, rT_v,
          hridx_v, tridx_v, oidx_v, hrow_v, trow_v, res_v,
          rel_s, sem0, sem1):
    wid = lax.axis_index("s") * NC + lax.axis_index("c")
    i16 = jnp.arange(LANES, dtype=jnp.int32)

    pltpu.sync_copy(rtyp, rt_v)
    pltpu.sync_copy(hids, hid_v)
    pltpu.sync_copy(tids, tid_v)

    for l in range(MLOC):
        m = wid + NW * l

        @pl.when(m < REL_N)
        def _stage(l=l, m=m):
            pltpu.sync_copy(went.at[pl.ds(m * DIM, DIM), :], rstage_v)

            def tr_body(j, _):
                jv = jnp.full((LANES,), j, jnp.int32)
                for b in range(DIM // LANES):
                    col = plsc.load_gather(rstage_v, [i16 + LANES * b, jv])
                    rT_v[pl.ds(l * (DIM * DIM) + j * DIM + LANES * b,
                               LANES)] = col
                return 0

            lax.fori_loop(0, DIM, tr_body, 0)

    def scan_body(i, nm):
        rv = rt_v[pl.ds(i * LANES, LANES)]
        match = lax.bitwise_and(rv, jnp.int32(NW - 1)) == wid
        plsc.store_compressed(midx_v.at[pl.ds(nm, LANES)],
                              i * LANES + i16, mask=match)
        return nm + jnp.sum(match.astype(jnp.int32))

    nm = lax.fori_loop(0, N // LANES, scan_body, jnp.int32(0))

    @pl.when(nm > 0)
    def _process():
        last = midx_v[pl.ds(nm - 1, LANES)][0]
        for b in range(CHUNK // LANES):
            midx_v[pl.ds(nm + LANES * b, LANES)] = jnp.full(
                (LANES,), last, jnp.int32)

        nchunks = lax.div(nm + (CHUNK - 1), jnp.int32(CHUNK))

        def chunk_body(c, _):
            cbase = c * CHUNK

            for g in range(CHUNK // LANES):
                m16 = midx_v[pl.ds(cbase + g * LANES, LANES)]
                hridx_v[pl.ds(g * LANES, LANES)] = plsc.load_gather(hid_v, [m16])
                tridx_v[pl.ds(g * LANES, LANES)] = plsc.load_gather(tid_v, [m16])
                base16 = lax.shift_left(
                    lax.shift_right_logical(plsc.load_gather(rt_v, [m16]), 5),
                    12)
                oidx_v[pl.ds(g * LANES, LANES)] = m16
                for k in range(LANES):
                    rel_s[g * LANES + k] = base16[k]

            cph = pltpu.async_copy(went.at[hridx_v], hrow_v, sem0)
            cpt = pltpu.async_copy(went.at[tridx_v], trow_v, sem1)
            cph.wait()
            cpt.wait()

            def grp_body(g, _2):
                def s_body(i, carry):
                    ppv, ptv, ttv = carry
                    s = g * LANES + i
                    base = rel_s[s]
                    hb = [hrow_v[s, pl.ds(LANES * b, LANES)] for b in range(4)]
                    acc = [jnp.zeros((LANES,), jnp.float32) for _ in range(4)]
                    for j in range(DIM):
                        hjv = jnp.full((LANES,), hb[j // LANES][j % LANES],
                                       jnp.float32)
                        for b in range(4):
                            acc[b] = acc[b] + rT_v[pl.ds(
                                base + j * DIM + LANES * b, LANES)] * hjv
                    tb = [trow_v[s, pl.ds(LANES * b, LANES)] for b in range(4)]
                    ppp = acc[0] * acc[0] + acc[1] * acc[1] + acc[2] * acc[2] + acc[3] * acc[3]
                    ptp = acc[0] * tb[0] + acc[1] * tb[1] + acc[2] * tb[2] + acc[3] * tb[3]
                    ttp = tb[0] * tb[0] + tb[1] * tb[1] + tb[2] * tb[2] + tb[3] * tb[3]
                    lm = i16 == i
                    ppv = jnp.where(lm, jnp.sum(ppp), ppv)
                    ptv = jnp.where(lm, jnp.sum(ptp), ptv)
                    ttv = jnp.where(lm, jnp.sum(ttp), ttv)
                    return ppv, ptv, ttv

                z = jnp.zeros((LANES,), jnp.float32)
                ppv, ptv, ttv = lax.fori_loop(0, LANES, s_body, (z, z, z))
                c16 = ptv * _rsqrt(jnp.maximum(ppv * ttv, 1e-30))
                v16 = jnp.maximum(2.0 - 2.0 * c16, 0.0)
                res_v[pl.ds(g * LANES, LANES)] = v16 * _rsqrt(
                    jnp.maximum(v16, 1e-30))
                return 0

            lax.fori_loop(0, CHUNK // LANES, grp_body, 0)

            pltpu.async_copy(res_v, out.at[oidx_v], sem0).wait()
            return 0

        lax.fori_loop(0, nchunks, chunk_body, 0)


def kernel(W_ent, W_rel, h_ids, r_typ, t_ids):
    mesh = plsc.VectorSubcoreMesh(core_axis_name="c", subcore_axis_name="s",
                                  num_cores=NC, num_subcores=NS)
    f = pl.kernel(
        _body,
        out_type=jax.ShapeDtypeStruct((N,), jnp.float32),
        mesh=mesh,
        compiler_params=pltpu.CompilerParams(needs_layout_passes=False,
                                             use_tc_tiling_on_sc=False),
        scratch_types=[
            pltpu.VMEM((N,), jnp.int32),
            pltpu.VMEM((N,), jnp.int32),
            pltpu.VMEM((N,), jnp.int32),
            pltpu.VMEM((N + CHUNK,), jnp.int32),
            pltpu.VMEM((DIM, DIM), jnp.float32),
            pltpu.VMEM((MLOC * DIM * DIM,), jnp.float32),
            pltpu.VMEM((CHUNK,), jnp.int32),
            pltpu.VMEM((CHUNK,), jnp.int32),
            pltpu.VMEM((CHUNK,), jnp.int32),
            pltpu.VMEM((CHUNK, DIM), jnp.float32),
            pltpu.VMEM((CHUNK, DIM), jnp.float32),
            pltpu.VMEM((CHUNK,), jnp.float32),
            pltpu.SMEM((CHUNK,), jnp.int32),
            pltpu.SemaphoreType.DMA,
            pltpu.SemaphoreType.DMA,
        ],
    )
    return f(W_ent, h_ids, r_typ, t_ids)
